# Initial kernel scaffold; baseline (speedup 1.0000x reference)
import functools

import jax
import jax.numpy as jnp
from jax import lax
from jax.experimental import pallas as pl
from jax.experimental.pallas import tpu as pltpu
from jax.experimental.pallas import tpu_sc as plsc

_NC = 2
_NS = 16
_NW = _NC * _NS
_LANES = 16

_EK = 125
_DK = 80


def _make_agg(n_nodes, n_edges, feat):
  epw = n_edges // _NW
  nch = epw // _EK
  rpt = n_nodes // _NS
  mesh = plsc.VectorSubcoreMesh(core_axis_name="c", subcore_axis_name="s")

  @functools.partial(
      pl.kernel,
      mesh=mesh,
      out_type=(
          jax.ShapeDtypeStruct((2 * n_nodes, feat), jnp.float32),
          jax.ShapeDtypeStruct((2 * n_nodes, 16), jnp.float32),
      ),
      scratch_types=[
          pltpu.VMEM((nch, _EK), jnp.int32),
          pltpu.VMEM((nch, _EK), jnp.int32),
          pltpu.VMEM((_EK, feat), jnp.float32),
          pltpu.VMEM((_EK, 16), jnp.float32),
          pltpu.VMEM_SHARED((n_nodes, feat), jnp.float32),
          pltpu.VMEM_SHARED((n_nodes, 16), jnp.float32),
          pltpu.SemaphoreType.DMA,
      ],
  )
  def agg(table, srci, dsti, zfeat, zcnt, ones_in, out_sum, out_cnt,
          sidx, didx, rows, ones_v, acc, cnt, sem):
    c = lax.axis_index("c")
    s = lax.axis_index("s")
    wid = s * _NC + c
    r0 = s * rpt
    pltpu.sync_copy(zfeat.at[pl.ds(r0, rpt)], acc.at[pl.ds(r0, rpt)])
    pltpu.sync_copy(zcnt.at[pl.ds(r0, rpt)], cnt.at[pl.ds(r0, rpt)])
    pltpu.sync_copy(ones_in, ones_v)
    pltpu.sync_copy(srci.at[wid], sidx)
    pltpu.sync_copy(dsti.at[wid], didx)
    plsc.subcore_barrier()

    def body(ci, carry):
      pltpu.async_copy(table.at[sidx.at[ci]], rows, sem).wait()
      pltpu.sync_copy(rows, acc.at[didx.at[ci]], add=True)
      pltpu.sync_copy(ones_v, cnt.at[didx.at[ci]], add=True)
      return carry

    lax.fori_loop(0, nch, body, 0)
    plsc.subcore_barrier()
    pltpu.sync_copy(acc.at[pl.ds(r0, rpt)],
                    out_sum.at[pl.ds(c * n_nodes + r0, rpt)])
    pltpu.sync_copy(cnt.at[pl.ds(r0, rpt)],
                    out_cnt.at[pl.ds(c * n_nodes + r0, rpt)])

  return agg


def _make_decode(n_nodes, n_pairs, feat):
  ppw = n_pairs // _NW
  nch = ppw // _DK
  ngrp = _DK // _LANES
  scale = 1.0 / (feat ** 0.5)
  mesh = plsc.VectorSubcoreMesh(core_axis_name="c", subcore_axis_name="s")

  @functools.partial(
      pl.kernel,
      mesh=mesh,
      out_type=jax.ShapeDtypeStruct((_NW, nch, _DK), jnp.float32),
      scratch_types=[
          pltpu.VMEM((nch, _DK), jnp.int32),
          pltpu.VMEM((nch, _DK), jnp.int32),
          pltpu.VMEM((_DK, feat), jnp.float32),
          pltpu.VMEM((_DK, feat), jnp.float32),
          pltpu.VMEM((_LANES, _LANES), jnp.float32),
          pltpu.VMEM((nch, _DK), jnp.float32),
          pltpu.SemaphoreType.DMA,
          pltpu.SemaphoreType.DMA,
      ],
  )
  def dec(z, aidx_h, bidx_h, out, aidx, bidx, ra, rb, t, outv, sema, semb):
    c = lax.axis_index("c")
    s = lax.axis_index("s")
    wid = s * _NC + c
    pltpu.sync_copy(aidx_h.at[wid], aidx)
    pltpu.sync_copy(bidx_h.at[wid], bidx)
    lanes = lax.iota(jnp.int32, 16)

    def chunk(ci, carry):
      pltpu.async_copy(z.at[aidx.at[ci]], ra, sema).wait()
      pltpu.async_copy(z.at[bidx.at[ci]], rb, semb).wait()

      def group(g, carry2):
        base = g * _LANES
        for e in range(_LANES):
          accv = ra[base + e, pl.ds(0, 16)] * rb[base + e, pl.ds(0, 16)]
          for j in range(1, feat // 16):
            accv = accv + (ra[base + e, pl.ds(16 * j, 16)] *
                           rb[base + e, pl.ds(16 * j, 16)])
          t[e, :] = accv
        dots = jnp.zeros((16,), jnp.float32)
        for l in range(_LANES):
          dots = dots + plsc.load_gather(
              t, [lanes, jnp.full((16,), l, jnp.int32)])
        outv[ci, pl.ds(base, 16)] = dots * scale
        return carry2

      lax.fori_loop(0, ngrp, group, 0)
      return carry

    lax.fori_loop(0, nch, chunk, 0)
    pltpu.sync_copy(outv, out.at[wid])

  return dec


def _proj_body(x_ref, wrel_ref, wroot_ref, b_ref, p_ref, r_ref):
  xb = x_ref[...]
  p_ref[...] = jnp.dot(xb, wrel_ref[...], preferred_element_type=jnp.float32)
  r_ref[...] = (jnp.dot(xb, wroot_ref[...], preferred_element_type=jnp.float32)
                + b_ref[...])


def _mid_body(s0_ref, s1_ref, c0_ref, c1_ref, r1_ref, wrel_ref, wroot_ref,
              b_ref, p2_ref, r2_ref):
  cnt = jnp.maximum(c0_ref[...][:, 0:1] + c1_ref[...][:, 0:1], 1.0)
  mean = (s0_ref[...] + s1_ref[...]) / cnt
  z = jnp.maximum(mean + r1_ref[...], 0.0)
  p2_ref[...] = jnp.dot(z, wrel_ref[...], preferred_element_type=jnp.float32)
  r2_ref[...] = (jnp.dot(z, wroot_ref[...], preferred_element_type=jnp.float32)
                 + b_ref[...])


def _fin_body(s0_ref, s1_ref, c0_ref, c1_ref, r2_ref, z2_ref):
  cnt = jnp.maximum(c0_ref[...][:, 0:1] + c1_ref[...][:, 0:1], 1.0)
  z2_ref[...] = (s0_ref[...] + s1_ref[...]) / cnt + r2_ref[...]


def kernel(x, edge_index, edge_label_index, W1_rel, b1_rel, W1_root,
           W2_rel, b2_rel, W2_root):
  n, in_dim = x.shape
  h = W1_rel.shape[0]
  e = edge_index.shape[1]
  el = edge_label_index.shape[1]
  f32 = jnp.float32

  epw = e // _NW
  src = edge_index[0].reshape(_NW, epw // _EK, _EK)
  dst = edge_index[1].reshape(_NW, epw // _EK, _EK)
  ppw = el // _NW
  la = edge_label_index[0].reshape(_NW, ppw // _DK, _DK)
  lb = edge_label_index[1].reshape(_NW, ppw // _DK, _DK)

  zfeat = jnp.zeros((n, h), f32)
  zcnt = jnp.zeros((n, 16), f32)
  ones = jnp.ones((_EK, 16), f32)

  rblk = 1000
  grid = (n // rblk,)
  full128 = pl.BlockSpec((in_dim, h), lambda i: (0, 0))
  bias = pl.BlockSpec((1, h), lambda i: (0, 0))
  row = pl.BlockSpec((rblk, h), lambda i: (i, 0))
  rowc = pl.BlockSpec((rblk, 16), lambda i: (i, 0))
  row_hi = pl.BlockSpec((rblk, h), lambda i: (i + n // rblk, 0))
  rowc_hi = pl.BlockSpec((rblk, 16), lambda i: (i + n // rblk, 0))
  sds = lambda shape: jax.ShapeDtypeStruct(shape, f32)

  proj = pl.pallas_call(
      _proj_body, grid=grid,
      in_specs=[row, full128, full128, bias],
      out_specs=[row, row],
      out_shape=[sds((n, h)), sds((n, h))],
  )
  mid = pl.pallas_call(
      _mid_body, grid=grid,
      in_specs=[row, row_hi, rowc, rowc_hi, row, full128, full128, bias],
      out_specs=[row, row],
      out_shape=[sds((n, h)), sds((n, h))],
  )
  fin = pl.pallas_call(
      _fin_body, grid=grid,
      in_specs=[row, row_hi, rowc, rowc_hi, row],
      out_specs=row,
      out_shape=sds((n, h)),
  )

  agg = _make_agg(n, e, h)
  dec = _make_decode(n, el, h)

  w1rel_t = W1_rel.T
  w1root_t = W1_root.T
  w2rel_t = W2_rel.T
  w2root_t = W2_root.T
  b1 = b1_rel.reshape(1, h)
  b2 = b2_rel.reshape(1, h)

  p1, r1 = proj(x, w1rel_t, w1root_t, b1)
  s1, c1 = agg(p1, src, dst, zfeat, zcnt, ones)
  p2, r2 = mid(s1, c1, r1, w2rel_t, w2root_t, b2)
  s2, c2 = agg(p2, src, dst, zfeat, zcnt, ones)
  z2 = fin(s2, c2, r2)
  out = dec(z2, la, lb)
  return out.reshape(el)


# speedup vs baseline: 4.4937x; 4.4937x over previous
import functools

import jax
import jax.numpy as jnp
from jax import lax
from jax.experimental import pallas as pl
from jax.experimental.pallas import tpu as pltpu
from jax.experimental.pallas import tpu_sc as plsc

_NC = 2
_NS = 16
_NW = _NC * _NS
_LANES = 16

_EK = 125
_IG = 8
_DK = 80
_FB = 104


def _make_agg(n_nodes, n_edges, feat):
  epw = n_edges // _NW
  nch = epw // _EK
  rpt = (n_nodes // _NS) & ~7
  tail = n_nodes - _NS * rpt
  nfl = rpt // _FB
  assert nfl * _FB == rpt and tail <= _FB
  mesh = plsc.VectorSubcoreMesh(core_axis_name="c", subcore_axis_name="s")

  @functools.partial(
      pl.kernel,
      mesh=mesh,
      out_type=jax.ShapeDtypeStruct((2 * n_nodes, feat), jnp.float32),
      scratch_types=[
          pltpu.VMEM((_IG, _EK), jnp.int32),
          pltpu.VMEM((_IG, _EK), jnp.int32),
          pltpu.VMEM((_EK, feat), jnp.float32),
          pltpu.VMEM_SHARED((n_nodes, feat), jnp.float32),
          pltpu.SemaphoreType.DMA,
      ],
  )
  def agg(table, srci, dsti, zfeat, out_sum, sidx, didx, rows, acc, sem):
    c = lax.axis_index("c")
    s = lax.axis_index("s")
    wid = s * _NC + c
    r0 = s * rpt
    pltpu.sync_copy(zfeat, rows.at[pl.ds(0, _FB)])
    for k in range(nfl):
      pltpu.sync_copy(rows.at[pl.ds(0, _FB)], acc.at[pl.ds(r0 + k * _FB, _FB)])
    if tail:
      @pl.when(s == 0)
      def _zero_tail():
        t0 = _NS * rpt
        pltpu.sync_copy(rows.at[pl.ds(0, tail)], acc.at[pl.ds(t0, tail)])
    plsc.subcore_barrier()

    def group_body(g, carry):
      g0 = wid * nch + g * _IG
      pltpu.sync_copy(srci.at[pl.ds(g0, _IG)], sidx)
      pltpu.sync_copy(dsti.at[pl.ds(g0, _IG)], didx)

      def body(ci, carry2):
        pltpu.async_copy(table.at[sidx.at[ci]], rows, sem).wait()
        pltpu.sync_copy(rows, acc.at[didx.at[ci]], add=True)
        return carry2

      lax.fori_loop(0, _IG, body, 0)
      return carry

    lax.fori_loop(0, nch // _IG, group_body, 0)
    plsc.subcore_barrier()
    for k in range(nfl):
      pltpu.sync_copy(acc.at[pl.ds(r0 + k * _FB, _FB)], rows.at[pl.ds(0, _FB)])
      pltpu.sync_copy(rows.at[pl.ds(0, _FB)],
                      out_sum.at[pl.ds(c * n_nodes + r0 + k * _FB, _FB)])
    if tail:
      @pl.when(s == 0)
      def _flush_tail():
        t0 = _NS * rpt
        pltpu.sync_copy(acc.at[pl.ds(t0, tail)], rows.at[pl.ds(0, tail)])
        pltpu.sync_copy(rows.at[pl.ds(0, tail)],
                        out_sum.at[pl.ds(c * n_nodes + t0, tail)])

  return agg


def _make_cnt(n_nodes, n_edges, feat):
  epw = n_edges // _NW
  nch = epw // _EK
  rpt = (n_nodes // _NS) & ~7
  tail = n_nodes - _NS * rpt
  nfl = rpt // _FB
  assert nfl * _FB == rpt and tail <= _FB
  mesh = plsc.VectorSubcoreMesh(core_axis_name="c", subcore_axis_name="s")

  @functools.partial(
      pl.kernel,
      mesh=mesh,
      out_type=jax.ShapeDtypeStruct((2 * n_nodes, feat), jnp.float32),
      scratch_types=[
          pltpu.VMEM((_IG, _EK), jnp.int32),
          pltpu.VMEM((_EK, feat), jnp.float32),
          pltpu.VMEM_SHARED((n_nodes, feat), jnp.float32),
      ],
  )
  def cntk(dsti, zfeat, ones_in, out_cnt, didx, ones_v, cnt):
    c = lax.axis_index("c")
    s = lax.axis_index("s")
    wid = s * _NC + c
    r0 = s * rpt
    pltpu.sync_copy(zfeat, ones_v.at[pl.ds(0, _FB)])
    for k in range(nfl):
      pltpu.sync_copy(ones_v.at[pl.ds(0, _FB)],
                      cnt.at[pl.ds(r0 + k * _FB, _FB)])
    if tail:
      @pl.when(s == 0)
      def _zero_tail():
        t0 = _NS * rpt
        pltpu.sync_copy(ones_v.at[pl.ds(0, tail)], cnt.at[pl.ds(t0, tail)])
    pltpu.sync_copy(ones_in, ones_v)
    plsc.subcore_barrier()

    def group_body(g, carry):
      g0 = wid * nch + g * _IG
      pltpu.sync_copy(dsti.at[pl.ds(g0, _IG)], didx)

      def body(ci, carry2):
        pltpu.sync_copy(ones_v, cnt.at[didx.at[ci]], add=True)
        return carry2

      lax.fori_loop(0, _IG, body, 0)
      return carry

    lax.fori_loop(0, nch // _IG, group_body, 0)
    plsc.subcore_barrier()
    for k in range(nfl):
      pltpu.sync_copy(cnt.at[pl.ds(r0 + k * _FB, _FB)],
                      ones_v.at[pl.ds(0, _FB)])
      pltpu.sync_copy(ones_v.at[pl.ds(0, _FB)],
                      out_cnt.at[pl.ds(c * n_nodes + r0 + k * _FB, _FB)])
    if tail:
      @pl.when(s == 0)
      def _flush_tail():
        t0 = _NS * rpt
        pltpu.sync_copy(cnt.at[pl.ds(t0, tail)], ones_v.at[pl.ds(0, tail)])
        pltpu.sync_copy(ones_v.at[pl.ds(0, tail)],
                        out_cnt.at[pl.ds(c * n_nodes + t0, tail)])

  return cntk


def _make_decode(n_nodes, n_pairs, feat):
  ppw = n_pairs // _NW
  nch = ppw // _DK
  ngrp = _DK // _LANES
  scale = 1.0 / (feat ** 0.5)
  mesh = plsc.VectorSubcoreMesh(core_axis_name="c", subcore_axis_name="s")

  @functools.partial(
      pl.kernel,
      mesh=mesh,
      out_type=jax.ShapeDtypeStruct((_NW, nch, _DK), jnp.float32),
      scratch_types=[
          pltpu.VMEM((nch, _DK), jnp.int32),
          pltpu.VMEM((nch, _DK), jnp.int32),
          pltpu.VMEM((_DK, feat), jnp.float32),
          pltpu.VMEM((_DK, feat), jnp.float32),
          pltpu.VMEM((nch, _DK), jnp.float32),
          pltpu.SemaphoreType.DMA,
          pltpu.SemaphoreType.DMA,
      ],
  )
  def dec(z, aidx_h, bidx_h, out, aidx, bidx, ra, rb, outv, sema, semb):
    c = lax.axis_index("c")
    s = lax.axis_index("s")
    wid = s * _NC + c
    pltpu.sync_copy(aidx_h.at[wid], aidx)
    pltpu.sync_copy(bidx_h.at[wid], bidx)
    lanes = lax.iota(jnp.int32, 16)
    rot_idx = [(lanes + sh) & (_LANES - 1) for sh in (8, 4, 2, 1)]

    def chunk(ci, carry):
      pltpu.async_copy(z.at[aidx.at[ci]], ra, sema).wait()
      pltpu.async_copy(z.at[bidx.at[ci]], rb, semb).wait()

      def group(g, carry2):
        base = g * _LANES
        dots = jnp.zeros((16,), jnp.float32)
        for e in range(_LANES):
          accv = ra[base + e, pl.ds(0, 16)] * rb[base + e, pl.ds(0, 16)]
          for j in range(1, feat // 16):
            accv = accv + (ra[base + e, pl.ds(16 * j, 16)] *
                           rb[base + e, pl.ds(16 * j, 16)])
          for ri in rot_idx:
            accv = accv + accv.at[ri].get(mode="promise_in_bounds")
          dots = dots + jnp.where(lanes == e, accv, 0.0)
        outv[ci, pl.ds(base, 16)] = dots * scale
        return carry2

      lax.fori_loop(0, ngrp, group, 0)
      return carry

    lax.fori_loop(0, nch, chunk, 0)
    pltpu.sync_copy(outv, out.at[wid])

  return dec


def _proj_body(x_ref, wrel_ref, wroot_ref, b_ref, p_ref, r_ref):
  xb = x_ref[...]
  p_ref[...] = jnp.dot(xb, wrel_ref[...], preferred_element_type=jnp.float32)
  r_ref[...] = (jnp.dot(xb, wroot_ref[...], preferred_element_type=jnp.float32)
                + b_ref[...])


def _mid_body(s0_ref, s1_ref, c0_ref, c1_ref, r1_ref, wrel_ref, wroot_ref,
              b_ref, p2_ref, r2_ref, inv_ref):
  cnt = c0_ref[...][:, 0:1] + c1_ref[...][:, 0:1]
  inv = 1.0 / jnp.maximum(cnt, 1.0)
  mean = (s0_ref[...] + s1_ref[...]) * inv
  z = jnp.maximum(mean + r1_ref[...], 0.0)
  p2_ref[...] = jnp.dot(z, wrel_ref[...], preferred_element_type=jnp.float32)
  r2_ref[...] = (jnp.dot(z, wroot_ref[...], preferred_element_type=jnp.float32)
                 + b_ref[...])
  inv_ref[...] = jnp.broadcast_to(inv, (inv.shape[0], 16))


def _fin_body(s0_ref, s1_ref, inv_ref, r2_ref, z2_ref):
  z2_ref[...] = ((s0_ref[...] + s1_ref[...]) * inv_ref[...][:, 0:1]
                 + r2_ref[...])


def kernel(x, edge_index, edge_label_index, W1_rel, b1_rel, W1_root,
           W2_rel, b2_rel, W2_root):
  n, in_dim = x.shape
  h = W1_rel.shape[0]
  e = edge_index.shape[1]
  el = edge_label_index.shape[1]
  f32 = jnp.float32

  epw = e // _NW
  src = edge_index[0].reshape(_NW * (epw // _EK), _EK)
  dst = edge_index[1].reshape(_NW * (epw // _EK), _EK)
  ppw = el // _NW
  la = edge_label_index[0].reshape(_NW, ppw // _DK, _DK)
  lb = edge_label_index[1].reshape(_NW, ppw // _DK, _DK)

  zfeat = jnp.zeros((_FB, h), f32)
  ones = jnp.ones((_EK, h), f32)

  rblk = 1000
  grid = (n // rblk,)
  full128 = pl.BlockSpec((in_dim, h), lambda i: (0, 0))
  bias = pl.BlockSpec((1, h), lambda i: (0, 0))
  row = pl.BlockSpec((rblk, h), lambda i: (i, 0))
  rowc = pl.BlockSpec((rblk, 16), lambda i: (i, 0))
  row_hi = pl.BlockSpec((rblk, h), lambda i: (i + n // rblk, 0))
  sds = lambda shape: jax.ShapeDtypeStruct(shape, f32)

  proj = pl.pallas_call(
      _proj_body, grid=grid,
      in_specs=[row, full128, full128, bias],
      out_specs=[row, row],
      out_shape=[sds((n, h)), sds((n, h))],
  )
  mid = pl.pallas_call(
      _mid_body, grid=grid,
      in_specs=[row, row_hi, row, row_hi, row, full128, full128, bias],
      out_specs=[row, row, rowc],
      out_shape=[sds((n, h)), sds((n, h)), sds((n, 16))],
  )
  fin = pl.pallas_call(
      _fin_body, grid=grid,
      in_specs=[row, row_hi, rowc, row],
      out_specs=row,
      out_shape=sds((n, h)),
  )

  agg = _make_agg(n, e, h)
  cntk = _make_cnt(n, e, h)
  dec = _make_decode(n, el, h)

  w1rel_t = W1_rel.T
  w1root_t = W1_root.T
  w2rel_t = W2_rel.T
  w2root_t = W2_root.T
  b1 = b1_rel.reshape(1, h)
  b2 = b2_rel.reshape(1, h)

  p1, r1 = proj(x, w1rel_t, w1root_t, b1)
  c1 = cntk(dst, zfeat, ones)
  s1 = agg(p1, src, dst, zfeat)
  p2, r2, invc = mid(s1, s1, c1, c1, r1, w2rel_t, w2root_t, b2)
  s2 = agg(p2, src, dst, zfeat)
  z2 = fin(s2, s2, invc, r2)
  out = dec(z2, la, lb)
  return out.reshape(el)


# speedup vs baseline: 4.8628x; 1.0821x over previous
import functools

import jax
import jax.numpy as jnp
from jax import lax
from jax.experimental import pallas as pl
from jax.experimental.pallas import tpu as pltpu
from jax.experimental.pallas import tpu_sc as plsc

_NC = 2
_NS = 16
_NW = _NC * _NS
_LANES = 16

_EK = 125
_IG = 8
_DK = 80
_FB = 104


def _make_agg(n_nodes, n_edges, feat):
  epw = n_edges // _NW
  nch = epw // _EK
  rpt = (n_nodes // _NS) & ~7
  tail = n_nodes - _NS * rpt
  nfl = rpt // _FB
  assert nfl * _FB == rpt and tail <= _FB
  mesh = plsc.VectorSubcoreMesh(core_axis_name="c", subcore_axis_name="s")

  @functools.partial(
      pl.kernel,
      mesh=mesh,
      out_type=jax.ShapeDtypeStruct((2 * n_nodes, feat), jnp.float32),
      scratch_types=[
          pltpu.VMEM((_IG, _EK), jnp.int32),
          pltpu.VMEM((_IG, _EK), jnp.int32),
          pltpu.VMEM((_EK, feat), jnp.float32),
          pltpu.VMEM((_EK, feat), jnp.float32),
          pltpu.VMEM_SHARED((n_nodes, feat), jnp.float32),
          pltpu.SemaphoreType.DMA,
          pltpu.SemaphoreType.DMA,
      ],
  )
  def agg(table, srci, dsti, zfeat, out_sum,
          sidx, didx, rows, rows_b, acc, sema, semb):
    c = lax.axis_index("c")
    s = lax.axis_index("s")
    wid = s * _NC + c
    r0 = s * rpt
    pltpu.sync_copy(zfeat, rows.at[pl.ds(0, _FB)])
    for k in range(nfl):
      pltpu.sync_copy(rows.at[pl.ds(0, _FB)], acc.at[pl.ds(r0 + k * _FB, _FB)])
    if tail:
      @pl.when(s == 0)
      def _zero_tail():
        t0 = _NS * rpt
        pltpu.sync_copy(rows.at[pl.ds(0, tail)], acc.at[pl.ds(t0, tail)])
    plsc.subcore_barrier()

    def group_body(g, carry):
      g0 = wid * nch + g * _IG
      pltpu.sync_copy(srci.at[pl.ds(g0, _IG)], sidx)
      pltpu.sync_copy(dsti.at[pl.ds(g0, _IG)], didx)
      bufs = (rows, rows_b)
      sems = (sema, semb)
      handles = {0: pltpu.async_copy(table.at[sidx.at[0]], rows, sema)}
      for j in range(_IG):
        cur = bufs[j % 2]
        handles[j].wait()
        if j + 1 < _IG:
          handles[j + 1] = pltpu.async_copy(
              table.at[sidx.at[j + 1]], bufs[(j + 1) % 2], sems[(j + 1) % 2])
        pltpu.sync_copy(cur, acc.at[didx.at[j]], add=True)
      return carry

    lax.fori_loop(0, nch // _IG, group_body, 0)
    plsc.subcore_barrier()
    for k in range(nfl):
      pltpu.sync_copy(acc.at[pl.ds(r0 + k * _FB, _FB)], rows.at[pl.ds(0, _FB)])
      pltpu.sync_copy(rows.at[pl.ds(0, _FB)],
                      out_sum.at[pl.ds(c * n_nodes + r0 + k * _FB, _FB)])
    if tail:
      @pl.when(s == 0)
      def _flush_tail():
        t0 = _NS * rpt
        pltpu.sync_copy(acc.at[pl.ds(t0, tail)], rows.at[pl.ds(0, tail)])
        pltpu.sync_copy(rows.at[pl.ds(0, tail)],
                        out_sum.at[pl.ds(c * n_nodes + t0, tail)])

  return agg


def _make_cnt(n_nodes, n_edges, feat):
  epw = n_edges // _NW
  nch = epw // _EK
  rpt = (n_nodes // _NS) & ~7
  tail = n_nodes - _NS * rpt
  nfl = rpt // _FB
  assert nfl * _FB == rpt and tail <= _FB
  mesh = plsc.VectorSubcoreMesh(core_axis_name="c", subcore_axis_name="s")

  @functools.partial(
      pl.kernel,
      mesh=mesh,
      out_type=jax.ShapeDtypeStruct((2 * n_nodes, feat), jnp.float32),
      scratch_types=[
          pltpu.VMEM((_IG, _EK), jnp.int32),
          pltpu.VMEM((_EK, feat), jnp.float32),
          pltpu.VMEM_SHARED((n_nodes, feat), jnp.float32),
      ],
  )
  def cntk(dsti, zfeat, ones_in, out_cnt, didx, ones_v, cnt):
    c = lax.axis_index("c")
    s = lax.axis_index("s")
    wid = s * _NC + c
    r0 = s * rpt
    pltpu.sync_copy(zfeat, ones_v.at[pl.ds(0, _FB)])
    for k in range(nfl):
      pltpu.sync_copy(ones_v.at[pl.ds(0, _FB)],
                      cnt.at[pl.ds(r0 + k * _FB, _FB)])
    if tail:
      @pl.when(s == 0)
      def _zero_tail():
        t0 = _NS * rpt
        pltpu.sync_copy(ones_v.at[pl.ds(0, tail)], cnt.at[pl.ds(t0, tail)])
    pltpu.sync_copy(ones_in, ones_v)
    plsc.subcore_barrier()

    def group_body(g, carry):
      g0 = wid * nch + g * _IG
      pltpu.sync_copy(dsti.at[pl.ds(g0, _IG)], didx)

      def body(ci, carry2):
        pltpu.sync_copy(ones_v, cnt.at[didx.at[ci]], add=True)
        return carry2

      lax.fori_loop(0, _IG, body, 0)
      return carry

    lax.fori_loop(0, nch // _IG, group_body, 0)
    plsc.subcore_barrier()
    for k in range(nfl):
      pltpu.sync_copy(cnt.at[pl.ds(r0 + k * _FB, _FB)],
                      ones_v.at[pl.ds(0, _FB)])
      pltpu.sync_copy(ones_v.at[pl.ds(0, _FB)],
                      out_cnt.at[pl.ds(c * n_nodes + r0 + k * _FB, _FB)])
    if tail:
      @pl.when(s == 0)
      def _flush_tail():
        t0 = _NS * rpt
        pltpu.sync_copy(cnt.at[pl.ds(t0, tail)], ones_v.at[pl.ds(0, tail)])
        pltpu.sync_copy(ones_v.at[pl.ds(0, tail)],
                        out_cnt.at[pl.ds(c * n_nodes + t0, tail)])

  return cntk


def _make_decode(n_nodes, n_pairs, feat):
  ppw = n_pairs // _NW
  nch = ppw // _DK
  ngrp = _DK // _LANES
  scale = 1.0 / (feat ** 0.5)
  mesh = plsc.VectorSubcoreMesh(core_axis_name="c", subcore_axis_name="s")

  @functools.partial(
      pl.kernel,
      mesh=mesh,
      out_type=jax.ShapeDtypeStruct((_NW, nch, _DK), jnp.float32),
      scratch_types=[
          pltpu.VMEM((nch, _DK), jnp.int32),
          pltpu.VMEM((nch, _DK), jnp.int32),
          pltpu.VMEM((_DK, feat), jnp.float32),
          pltpu.VMEM((_DK, feat), jnp.float32),
          pltpu.VMEM((nch, _DK), jnp.float32),
          pltpu.SemaphoreType.DMA,
          pltpu.SemaphoreType.DMA,
      ],
  )
  def dec(z, aidx_h, bidx_h, out, aidx, bidx, ra, rb, outv, sema, semb):
    c = lax.axis_index("c")
    s = lax.axis_index("s")
    wid = s * _NC + c
    pltpu.sync_copy(aidx_h.at[wid], aidx)
    pltpu.sync_copy(bidx_h.at[wid], bidx)
    lanes = lax.iota(jnp.int32, 16)
    rot_idx = [(lanes + sh) & (_LANES - 1) for sh in (8, 4, 2, 1)]

    def chunk(ci, carry):
      pltpu.async_copy(z.at[aidx.at[ci]], ra, sema).wait()
      pltpu.async_copy(z.at[bidx.at[ci]], rb, semb).wait()

      def group(g, carry2):
        base = g * _LANES
        dots = jnp.zeros((16,), jnp.float32)
        for e in range(_LANES):
          accv = ra[base + e, pl.ds(0, 16)] * rb[base + e, pl.ds(0, 16)]
          for j in range(1, feat // 16):
            accv = accv + (ra[base + e, pl.ds(16 * j, 16)] *
                           rb[base + e, pl.ds(16 * j, 16)])
          for ri in rot_idx:
            accv = accv + accv.at[ri].get(mode="promise_in_bounds")
          dots = dots + jnp.where(lanes == e, accv, 0.0)
        outv[ci, pl.ds(base, 16)] = dots * scale
        return carry2

      lax.fori_loop(0, ngrp, group, 0)
      return carry

    lax.fori_loop(0, nch, chunk, 0)
    pltpu.sync_copy(outv, out.at[wid])

  return dec


def _proj_body(x_ref, wrel_ref, wroot_ref, b_ref, p_ref, r_ref):
  xb = x_ref[...]
  p_ref[...] = jnp.dot(xb, wrel_ref[...], preferred_element_type=jnp.float32)
  r_ref[...] = (jnp.dot(xb, wroot_ref[...], preferred_element_type=jnp.float32)
                + b_ref[...])


def _mid_body(s0_ref, s1_ref, c0_ref, c1_ref, r1_ref, wrel_ref, wroot_ref,
              b_ref, p2_ref, r2_ref, inv_ref):
  cnt = c0_ref[...][:, 0:1] + c1_ref[...][:, 0:1]
  inv = 1.0 / jnp.maximum(cnt, 1.0)
  mean = (s0_ref[...] + s1_ref[...]) * inv
  z = jnp.maximum(mean + r1_ref[...], 0.0)
  p2_ref[...] = jnp.dot(z, wrel_ref[...], preferred_element_type=jnp.float32)
  r2_ref[...] = (jnp.dot(z, wroot_ref[...], preferred_element_type=jnp.float32)
                 + b_ref[...])
  inv_ref[...] = jnp.broadcast_to(inv, (inv.shape[0], 16))


def _fin_body(s0_ref, s1_ref, inv_ref, r2_ref, z2_ref):
  z2_ref[...] = ((s0_ref[...] + s1_ref[...]) * inv_ref[...][:, 0:1]
                 + r2_ref[...])


def kernel(x, edge_index, edge_label_index, W1_rel, b1_rel, W1_root,
           W2_rel, b2_rel, W2_root):
  n, in_dim = x.shape
  h = W1_rel.shape[0]
  e = edge_index.shape[1]
  el = edge_label_index.shape[1]
  f32 = jnp.float32

  epw = e // _NW
  src = edge_index[0].reshape(_NW * (epw // _EK), _EK)
  dst = edge_index[1].reshape(_NW * (epw // _EK), _EK)
  ppw = el // _NW
  la = edge_label_index[0].reshape(_NW, ppw // _DK, _DK)
  lb = edge_label_index[1].reshape(_NW, ppw // _DK, _DK)

  zfeat = jnp.zeros((_FB, h), f32)
  ones = jnp.ones((_EK, h), f32)

  rblk = 1000
  grid = (n // rblk,)
  full128 = pl.BlockSpec((in_dim, h), lambda i: (0, 0))
  bias = pl.BlockSpec((1, h), lambda i: (0, 0))
  row = pl.BlockSpec((rblk, h), lambda i: (i, 0))
  rowc = pl.BlockSpec((rblk, 16), lambda i: (i, 0))
  row_hi = pl.BlockSpec((rblk, h), lambda i: (i + n // rblk, 0))
  sds = lambda shape: jax.ShapeDtypeStruct(shape, f32)

  proj = pl.pallas_call(
      _proj_body, grid=grid,
      in_specs=[row, full128, full128, bias],
      out_specs=[row, row],
      out_shape=[sds((n, h)), sds((n, h))],
  )
  mid = pl.pallas_call(
      _mid_body, grid=grid,
      in_specs=[row, row_hi, row, row_hi, row, full128, full128, bias],
      out_specs=[row, row, rowc],
      out_shape=[sds((n, h)), sds((n, h)), sds((n, 16))],
  )
  fin = pl.pallas_call(
      _fin_body, grid=grid,
      in_specs=[row, row_hi, rowc, row],
      out_specs=row,
      out_shape=sds((n, h)),
  )

  agg = _make_agg(n, e, h)
  cntk = _make_cnt(n, e, h)
  dec = _make_decode(n, el, h)

  w1rel_t = W1_rel.T
  w1root_t = W1_root.T
  w2rel_t = W2_rel.T
  w2root_t = W2_root.T
  b1 = b1_rel.reshape(1, h)
  b2 = b2_rel.reshape(1, h)

  p1, r1 = proj(x, w1rel_t, w1root_t, b1)
  c1 = cntk(dst, zfeat, ones)
  s1 = agg(p1, src, dst, zfeat)
  p2, r2, invc = mid(s1, s1, c1, c1, r1, w2rel_t, w2root_t, b2)
  s2 = agg(p2, src, dst, zfeat)
  z2 = fin(s2, s2, invc, r2)
  out = dec(z2, la, lb)
  return out.reshape(el)


# speedup vs baseline: 4.8703x; 1.0015x over previous
import functools

import jax
import jax.numpy as jnp
from jax import lax
from jax.experimental import pallas as pl
from jax.experimental.pallas import tpu as pltpu
from jax.experimental.pallas import tpu_sc as plsc

_NC = 2
_NS = 16
_NW = _NC * _NS
_LANES = 16

_EK = 125
_IG = 8
_DK = 80
_FB = 104


def _make_agg(n_nodes, n_edges, feat):
  epw = n_edges // _NW
  nch = epw // _EK
  rpt = (n_nodes // _NS) & ~7
  tail = n_nodes - _NS * rpt
  nfl = rpt // _FB
  assert nfl * _FB == rpt and tail <= _FB
  mesh = plsc.VectorSubcoreMesh(core_axis_name="c", subcore_axis_name="s")

  @functools.partial(
      pl.kernel,
      mesh=mesh,
      out_type=jax.ShapeDtypeStruct((2 * n_nodes, feat), jnp.float32),
      scratch_types=[
          pltpu.VMEM((_IG, _EK), jnp.int32),
          pltpu.VMEM((_IG, _EK), jnp.int32),
          pltpu.VMEM((_EK, feat), jnp.float32),
          pltpu.VMEM((_EK, feat), jnp.float32),
          pltpu.VMEM_SHARED((n_nodes, feat), jnp.float32),
          pltpu.SemaphoreType.DMA,
          pltpu.SemaphoreType.DMA,
          pltpu.SemaphoreType.DMA,
          pltpu.SemaphoreType.DMA,
      ],
  )
  def agg(table, srci, dsti, zfeat, out_sum,
          sidx, didx, rows, rows_b, acc, sema, semb, sems_a, sems_b):
    c = lax.axis_index("c")
    s = lax.axis_index("s")
    wid = s * _NC + c
    r0 = s * rpt
    pltpu.sync_copy(zfeat, rows.at[pl.ds(0, _FB)])
    for k in range(nfl):
      pltpu.sync_copy(rows.at[pl.ds(0, _FB)], acc.at[pl.ds(r0 + k * _FB, _FB)])
    if tail:
      @pl.when(s == 0)
      def _zero_tail():
        t0 = _NS * rpt
        pltpu.sync_copy(rows.at[pl.ds(0, tail)], acc.at[pl.ds(t0, tail)])
    plsc.subcore_barrier()

    def group_body(g, carry):
      g0 = wid * nch + g * _IG
      pltpu.sync_copy(srci.at[pl.ds(g0, _IG)], sidx)
      pltpu.sync_copy(dsti.at[pl.ds(g0, _IG)], didx)
      bufs = (rows, rows_b)
      sems = (sema, semb)
      ssems = (sems_a, sems_b)
      gh = {0: pltpu.async_copy(table.at[sidx.at[0]], rows, sema)}
      sh = {}
      for j in range(_IG):
        cur = bufs[j % 2]
        gh[j].wait()
        if j + 1 < _IG:
          if j >= 1:
            sh[j - 1].wait()
          gh[j + 1] = pltpu.async_copy(
              table.at[sidx.at[j + 1]], bufs[(j + 1) % 2], sems[(j + 1) % 2])
        sh[j] = pltpu.async_copy(cur, acc.at[didx.at[j]], ssems[j % 2],
                                 add=True)
      sh[_IG - 2].wait()
      sh[_IG - 1].wait()
      return carry

    lax.fori_loop(0, nch // _IG, group_body, 0)
    plsc.subcore_barrier()
    for k in range(nfl):
      pltpu.sync_copy(acc.at[pl.ds(r0 + k * _FB, _FB)], rows.at[pl.ds(0, _FB)])
      pltpu.sync_copy(rows.at[pl.ds(0, _FB)],
                      out_sum.at[pl.ds(c * n_nodes + r0 + k * _FB, _FB)])
    if tail:
      @pl.when(s == 0)
      def _flush_tail():
        t0 = _NS * rpt
        pltpu.sync_copy(acc.at[pl.ds(t0, tail)], rows.at[pl.ds(0, tail)])
        pltpu.sync_copy(rows.at[pl.ds(0, tail)],
                        out_sum.at[pl.ds(c * n_nodes + t0, tail)])

  return agg


def _make_cnt(n_nodes, n_edges, feat):
  epw = n_edges // _NW
  nch = epw // _EK
  rpt = (n_nodes // _NS) & ~7
  tail = n_nodes - _NS * rpt
  nfl = rpt // _FB
  assert nfl * _FB == rpt and tail <= _FB
  mesh = plsc.VectorSubcoreMesh(core_axis_name="c", subcore_axis_name="s")

  @functools.partial(
      pl.kernel,
      mesh=mesh,
      out_type=jax.ShapeDtypeStruct((2 * n_nodes, feat), jnp.float32),
      scratch_types=[
          pltpu.VMEM((_IG, _EK), jnp.int32),
          pltpu.VMEM((_EK, feat), jnp.float32),
          pltpu.VMEM_SHARED((n_nodes, feat), jnp.float32),
      ],
  )
  def cntk(dsti, zfeat, ones_in, out_cnt, didx, ones_v, cnt):
    c = lax.axis_index("c")
    s = lax.axis_index("s")
    wid = s * _NC + c
    r0 = s * rpt
    pltpu.sync_copy(zfeat, ones_v.at[pl.ds(0, _FB)])
    for k in range(nfl):
      pltpu.sync_copy(ones_v.at[pl.ds(0, _FB)],
                      cnt.at[pl.ds(r0 + k * _FB, _FB)])
    if tail:
      @pl.when(s == 0)
      def _zero_tail():
        t0 = _NS * rpt
        pltpu.sync_copy(ones_v.at[pl.ds(0, tail)], cnt.at[pl.ds(t0, tail)])
    pltpu.sync_copy(ones_in, ones_v)
    plsc.subcore_barrier()

    def group_body(g, carry):
      g0 = wid * nch + g * _IG
      pltpu.sync_copy(dsti.at[pl.ds(g0, _IG)], didx)

      def body(ci, carry2):
        pltpu.sync_copy(ones_v, cnt.at[didx.at[ci]], add=True)
        return carry2

      lax.fori_loop(0, _IG, body, 0)
      return carry

    lax.fori_loop(0, nch // _IG, group_body, 0)
    plsc.subcore_barrier()
    for k in range(nfl):
      pltpu.sync_copy(cnt.at[pl.ds(r0 + k * _FB, _FB)],
                      ones_v.at[pl.ds(0, _FB)])
      pltpu.sync_copy(ones_v.at[pl.ds(0, _FB)],
                      out_cnt.at[pl.ds(c * n_nodes + r0 + k * _FB, _FB)])
    if tail:
      @pl.when(s == 0)
      def _flush_tail():
        t0 = _NS * rpt
        pltpu.sync_copy(cnt.at[pl.ds(t0, tail)], ones_v.at[pl.ds(0, tail)])
        pltpu.sync_copy(ones_v.at[pl.ds(0, tail)],
                        out_cnt.at[pl.ds(c * n_nodes + t0, tail)])

  return cntk


def _make_decode(n_nodes, n_pairs, feat):
  ppw = n_pairs // _NW
  nch = ppw // _DK
  ngrp = _DK // _LANES
  scale = 1.0 / (feat ** 0.5)
  mesh = plsc.VectorSubcoreMesh(core_axis_name="c", subcore_axis_name="s")

  @functools.partial(
      pl.kernel,
      mesh=mesh,
      out_type=jax.ShapeDtypeStruct((_NW, nch, _DK), jnp.float32),
      scratch_types=[
          pltpu.VMEM((nch, _DK), jnp.int32),
          pltpu.VMEM((nch, _DK), jnp.int32),
          pltpu.VMEM((_DK, feat), jnp.float32),
          pltpu.VMEM((_DK, feat), jnp.float32),
          pltpu.VMEM((nch, _DK), jnp.float32),
          pltpu.SemaphoreType.DMA,
          pltpu.SemaphoreType.DMA,
      ],
  )
  def dec(z, aidx_h, bidx_h, out, aidx, bidx, ra, rb, outv, sema, semb):
    c = lax.axis_index("c")
    s = lax.axis_index("s")
    wid = s * _NC + c
    pltpu.sync_copy(aidx_h.at[wid], aidx)
    pltpu.sync_copy(bidx_h.at[wid], bidx)
    lanes = lax.iota(jnp.int32, 16)
    rot_idx = [(lanes + sh) & (_LANES - 1) for sh in (8, 4, 2, 1)]

    def chunk(ci, carry):
      pltpu.async_copy(z.at[aidx.at[ci]], ra, sema).wait()
      pltpu.async_copy(z.at[bidx.at[ci]], rb, semb).wait()

      def group(g, carry2):
        base = g * _LANES
        dots = jnp.zeros((16,), jnp.float32)
        for e in range(_LANES):
          accv = ra[base + e, pl.ds(0, 16)] * rb[base + e, pl.ds(0, 16)]
          for j in range(1, feat // 16):
            accv = accv + (ra[base + e, pl.ds(16 * j, 16)] *
                           rb[base + e, pl.ds(16 * j, 16)])
          for ri in rot_idx:
            accv = accv + accv.at[ri].get(mode="promise_in_bounds")
          dots = dots + jnp.where(lanes == e, accv, 0.0)
        outv[ci, pl.ds(base, 16)] = dots * scale
        return carry2

      lax.fori_loop(0, ngrp, group, 0)
      return carry

    lax.fori_loop(0, nch, chunk, 0)
    pltpu.sync_copy(outv, out.at[wid])

  return dec


def _proj_body(x_ref, wrel_ref, wroot_ref, b_ref, p_ref, r_ref):
  xb = x_ref[...]
  p_ref[...] = jnp.dot(xb, wrel_ref[...], preferred_element_type=jnp.float32)
  r_ref[...] = (jnp.dot(xb, wroot_ref[...], preferred_element_type=jnp.float32)
                + b_ref[...])


def _mid_body(s0_ref, s1_ref, c0_ref, c1_ref, r1_ref, wrel_ref, wroot_ref,
              b_ref, p2_ref, r2_ref, inv_ref):
  cnt = c0_ref[...][:, 0:1] + c1_ref[...][:, 0:1]
  inv = 1.0 / jnp.maximum(cnt, 1.0)
  mean = (s0_ref[...] + s1_ref[...]) * inv
  z = jnp.maximum(mean + r1_ref[...], 0.0)
  p2_ref[...] = jnp.dot(z, wrel_ref[...], preferred_element_type=jnp.float32)
  r2_ref[...] = (jnp.dot(z, wroot_ref[...], preferred_element_type=jnp.float32)
                 + b_ref[...])
  inv_ref[...] = jnp.broadcast_to(inv, (inv.shape[0], 16))


def _fin_body(s0_ref, s1_ref, inv_ref, r2_ref, z2_ref):
  z2_ref[...] = ((s0_ref[...] + s1_ref[...]) * inv_ref[...][:, 0:1]
                 + r2_ref[...])


def kernel(x, edge_index, edge_label_index, W1_rel, b1_rel, W1_root,
           W2_rel, b2_rel, W2_root):
  n, in_dim = x.shape
  h = W1_rel.shape[0]
  e = edge_index.shape[1]
  el = edge_label_index.shape[1]
  f32 = jnp.float32

  epw = e // _NW
  src = edge_index[0].reshape(_NW * (epw // _EK), _EK)
  dst = edge_index[1].reshape(_NW * (epw // _EK), _EK)
  ppw = el // _NW
  la = edge_label_index[0].reshape(_NW, ppw // _DK, _DK)
  lb = edge_label_index[1].reshape(_NW, ppw // _DK, _DK)

  zfeat = jnp.zeros((_FB, h), f32)
  ones = jnp.ones((_EK, h), f32)

  rblk = 1000
  grid = (n // rblk,)
  full128 = pl.BlockSpec((in_dim, h), lambda i: (0, 0))
  bias = pl.BlockSpec((1, h), lambda i: (0, 0))
  row = pl.BlockSpec((rblk, h), lambda i: (i, 0))
  rowc = pl.BlockSpec((rblk, 16), lambda i: (i, 0))
  row_hi = pl.BlockSpec((rblk, h), lambda i: (i + n // rblk, 0))
  sds = lambda shape: jax.ShapeDtypeStruct(shape, f32)

  proj = pl.pallas_call(
      _proj_body, grid=grid,
      in_specs=[row, full128, full128, bias],
      out_specs=[row, row],
      out_shape=[sds((n, h)), sds((n, h))],
  )
  mid = pl.pallas_call(
      _mid_body, grid=grid,
      in_specs=[row, row_hi, row, row_hi, row, full128, full128, bias],
      out_specs=[row, row, rowc],
      out_shape=[sds((n, h)), sds((n, h)), sds((n, 16))],
  )
  fin = pl.pallas_call(
      _fin_body, grid=grid,
      in_specs=[row, row_hi, rowc, row],
      out_specs=row,
      out_shape=sds((n, h)),
  )

  agg = _make_agg(n, e, h)
  cntk = _make_cnt(n, e, h)
  dec = _make_decode(n, el, h)

  w1rel_t = W1_rel.T
  w1root_t = W1_root.T
  w2rel_t = W2_rel.T
  w2root_t = W2_root.T
  b1 = b1_rel.reshape(1, h)
  b2 = b2_rel.reshape(1, h)

  p1, r1 = proj(x, w1rel_t, w1root_t, b1)
  c1 = cntk(dst, zfeat, ones)
  s1 = agg(p1, src, dst, zfeat)
  p2, r2, invc = mid(s1, s1, c1, c1, r1, w2rel_t, w2root_t, b2)
  s2 = agg(p2, src, dst, zfeat)
  z2 = fin(s2, s2, invc, r2)
  out = dec(z2, la, lb)
  return out.reshape(el)
